# baseline (speedup 1.0000x reference)
import functools

import jax
import jax.numpy as jnp
from jax import lax
from jax.experimental import pallas as pl
from jax.experimental.pallas import tpu as pltpu
from jax.experimental.pallas import tpu_sc as plsc

LANES = 16
SUB = 128
TILE_R = 8
TILE_C = 128


@functools.cache
def _build(batch, seq, num_tables, vocab, d):
    info = plsc.get_sparse_core_info()
    nc, ns = info.num_cores, info.num_subcores
    nw = nc * ns
    s_out = seq // num_tables
    rows_per_tile = batch // nw
    per_tile = rows_per_tile * seq
    chunk = rows_per_tile * num_tables
    n_sub = chunk // SUB
    dblks = d // TILE_R
    assert batch % (nw * TILE_C) == 0 and rows_per_tile == TILE_C
    assert seq % num_tables == 0 and d % LANES == 0 and chunk % SUB == 0
    assert LANES % num_tables == 0 and s_out % 2 == 0

    mesh = plsc.VectorSubcoreMesh(core_axis_name="c", subcore_axis_name="s")

    @functools.partial(
        pl.kernel,
        mesh=mesh,
        compiler_params=pltpu.CompilerParams(use_tc_tiling_on_sc=False, needs_layout_passes=False),
        out_type=jax.ShapeDtypeStruct(
            (s_out * d * batch,), jnp.float32),
        scratch_types=[
            pltpu.VMEM((per_tile,), jnp.int32),
            pltpu.VMEM((2, n_sub, SUB), jnp.int32),
            pltpu.VMEM((2, chunk, d), jnp.float32),
            pltpu.VMEM((2, dblks * TILE_R * TILE_C), jnp.float32),
            pltpu.SemaphoreType.DMA,
            pltpu.SemaphoreType.DMA,
            pltpu.SemaphoreType.DMA,
            pltpu.SemaphoreType.DMA,
        ],
    )
    def k(ids_hbm, table_hbm, out_hbm, ids_v, idx_v, rows_v, slab_v,
          gsem0, gsem1, osem0, osem1):
        wid = lax.axis_index("s") * nc + lax.axis_index("c")
        base = wid * per_tile
        gsems = (gsem0, gsem1)
        osems = (osem0, osem1)

        pltpu.sync_copy(ids_hbm.at[pl.ds(base, per_tile)], ids_v)
        iota = lax.iota(jnp.int32, LANES)
        pat_ids = (iota // num_tables) * seq + iota % num_tables
        offs = (iota % num_tables) * vocab
        rowp = iota * num_tables

        def fire(c, b):
            for v in range(chunk // LANES):
                p = v * LANES
                g = plsc.load_gather(
                    ids_v, [pat_ids + (v * LANES // num_tables) * seq
                            + c * num_tables])
                idx_v[b, p // SUB, pl.ds(p % SUB, LANES)] = g + offs
            for si in range(n_sub):
                pltpu.async_copy(
                    table_hbm.at[idx_v.at[b, si]],
                    rows_v.at[b, pl.ds(si * SUB, SUB)],
                    gsems[b],
                )

        def drain(b):
            for si in range(n_sub):
                pltpu.make_async_copy(
                    table_hbm.at[idx_v.at[b, si]],
                    rows_v.at[b, pl.ds(si * SUB, SUB)],
                    gsems[b],
                ).wait()

        tpc = TILE_R * TILE_C

        s_stride = d * batch
        db_stride = (batch // TILE_C) * tpc

        def out_copies(c, b):
            return [
                pltpu.make_async_copy(
                    slab_v.at[b, pl.ds(db * tpc, tpc)],
                    out_hbm.at[pl.ds(c * s_stride + db * db_stride
                                     + wid * tpc, tpc)],
                    osems[b],
                )
                for db in range(dblks)
            ]

        def sum_store(c, b, first):
            if not first:
                for cp in out_copies(c, b):
                    cp.wait()

            rows2 = rows_v.at[b]
            slab1 = slab_v.at[b]

            @plsc.parallel_loop(0, TILE_C // LANES, step=1)
            def body(g):
                b16 = pl.multiple_of(g * LANES, LANES)
                rowvs = [rowp + (b16 * num_tables + i)
                         for i in range(num_tables)]

                @plsc.parallel_loop(0, LANES, step=1, unroll=4)
                def rbody(r):
                    rot = (iota + r) % LANES
                    sp = rot * TILE_C + iota
                    for dd in range(d // LANES):
                        colv = rot + dd * LANES
                        posv = sp + (dd * 2 * tpc + b16)
                        gs = [plsc.load_gather(rows2, [rowvs[i], colv])
                              for i in range(num_tables)]
                        while len(gs) > 1:
                            gs = [gs[j] + gs[j + 1]
                                  for j in range(0, len(gs) - 1, 2)] + (
                                      [gs[-1]] if len(gs) % 2 else [])
                        plsc.store_scatter(slab1, [posv], gs[0])
            for cp in out_copies(c, b):
                cp.start()

        fire(0, 0)
        fire(1, 1)

        def step(h, _):
            c = pl.multiple_of(h * 2, 2)
            drain(0)
            sum_store(c, 0, first=False)
            pl.when(c + 2 < s_out)(lambda: fire(c + 2, 0))
            drain(1)
            sum_store(c + 1, 1, first=False)
            pl.when(c + 3 < s_out)(lambda: fire(c + 3, 1))
            return 0

        drain(0)
        sum_store(0, 0, first=True)
        fire(2, 0)
        drain(1)
        sum_store(1, 1, first=True)
        fire(3, 1)
        lax.fori_loop(1, s_out // 2, step, 0)

        for b in range(2):
            for cp in out_copies(0, b):
                cp.wait()

    return k


def kernel(input_ids, tables):
    num_tables, vocab, d = tables.shape
    batch, seq = input_ids.shape
    s_out = seq // num_tables
    ids_flat = input_ids.reshape(-1)
    table_flat = tables.reshape(num_tables * vocab, d)
    raw = _build(batch, seq, num_tables, vocab, d)(ids_flat, table_flat)
    out = raw.reshape(s_out, d // TILE_R, batch // TILE_C, TILE_R, TILE_C)
    return out.transpose(2, 4, 0, 1, 3).reshape(batch, s_out, d)


# speedup vs baseline: 1.0318x; 1.0318x over previous
import functools

import jax
import jax.numpy as jnp
from jax import lax
from jax.experimental import pallas as pl
from jax.experimental.pallas import tpu as pltpu
from jax.experimental.pallas import tpu_sc as plsc

LANES = 16
SUB = 128
TILE_R = 8
TILE_C = 128


@functools.cache
def _build(batch, seq, num_tables, vocab, d):
    info = plsc.get_sparse_core_info()
    nc, ns = info.num_cores, info.num_subcores
    nw = nc * ns
    s_out = seq // num_tables
    rows_per_tile = batch // nw
    per_tile = rows_per_tile * seq
    chunk = rows_per_tile * num_tables
    n_sub = chunk // SUB
    dblks = d // TILE_R
    assert batch % (nw * TILE_C) == 0 and rows_per_tile == TILE_C
    assert seq % num_tables == 0 and d % LANES == 0 and chunk % SUB == 0
    assert LANES % num_tables == 0 and s_out % 2 == 0

    mesh = plsc.VectorSubcoreMesh(core_axis_name="c", subcore_axis_name="s")

    @functools.partial(
        pl.kernel,
        mesh=mesh,
        compiler_params=pltpu.CompilerParams(use_tc_tiling_on_sc=False, needs_layout_passes=False),
        out_type=jax.ShapeDtypeStruct(
            (s_out * d * batch,), jnp.float32),
        scratch_types=[
            pltpu.VMEM((per_tile,), jnp.int32),
            pltpu.VMEM((2, n_sub, SUB), jnp.int32),
            pltpu.VMEM((2, chunk, d), jnp.float32),
            pltpu.VMEM((2, dblks * TILE_R * TILE_C), jnp.float32),
            pltpu.SemaphoreType.DMA,
            pltpu.SemaphoreType.DMA,
            pltpu.SemaphoreType.DMA,
            pltpu.SemaphoreType.DMA,
        ],
    )
    def k(ids_hbm, table_hbm, out_hbm, ids_v, idx_v, rows_v, slab_v,
          gsem0, gsem1, osem0, osem1):
        wid = lax.axis_index("s") * nc + lax.axis_index("c")
        base = wid * per_tile
        gsems = (gsem0, gsem1)
        osems = (osem0, osem1)

        pltpu.sync_copy(ids_hbm.at[pl.ds(base, per_tile)], ids_v)
        iota = lax.iota(jnp.int32, LANES)
        pat_ids = (iota // num_tables) * seq + iota % num_tables
        offs = (iota % num_tables) * vocab
        rowp = iota * num_tables

        def fire(c, b):
            for v in range(chunk // LANES):
                p = v * LANES
                g = plsc.load_gather(
                    ids_v, [pat_ids + (v * LANES // num_tables) * seq
                            + c * num_tables])
                idx_v[b, p // SUB, pl.ds(p % SUB, LANES)] = g + offs
            for si in range(n_sub):
                pltpu.async_copy(
                    table_hbm.at[idx_v.at[b, si]],
                    rows_v.at[b, pl.ds(si * SUB, SUB)],
                    gsems[b],
                )

        def drain(b):
            for si in range(n_sub):
                pltpu.make_async_copy(
                    table_hbm.at[idx_v.at[b, si]],
                    rows_v.at[b, pl.ds(si * SUB, SUB)],
                    gsems[b],
                ).wait()

        tpc = TILE_R * TILE_C

        s_stride = d * batch
        db_stride = (batch // TILE_C) * tpc

        def out_copies(c, b):
            return [
                pltpu.make_async_copy(
                    slab_v.at[b, pl.ds(db * tpc, tpc)],
                    out_hbm.at[pl.ds(c * s_stride + db * db_stride
                                     + wid * tpc, tpc)],
                    osems[b],
                )
                for db in range(dblks)
            ]

        def sum_store(c, b, first):
            if not first:
                for cp in out_copies(c, b):
                    cp.wait()

            rows2 = rows_v.at[b]
            slab1 = slab_v.at[b]

            @plsc.parallel_loop(0, TILE_C // LANES, step=1)
            def body(g):
                b16 = pl.multiple_of(g * LANES, LANES)
                rowvs = [rowp + (b16 * num_tables + i)
                         for i in range(num_tables)]

                @plsc.parallel_loop(0, LANES, step=1, unroll=4)
                def rbody(r):
                    rot = (iota + r) % LANES
                    sp = ((rot // TILE_R) * (TILE_R * TILE_C)
                          + (rot % TILE_R) * TILE_C + iota)
                    for dd in range(d // LANES):
                        colv = rot + dd * LANES
                        posv = sp + (dd * 2 * tpc + b16)
                        acc = plsc.load_gather(rows2, [rowvs[0], colv])
                        for i in range(1, num_tables):
                            acc = acc + plsc.load_gather(
                                rows2, [rowvs[i], colv])
                        plsc.store_scatter(slab1, [posv], acc)
            for cp in out_copies(c, b):
                cp.start()

        fire(0, 0)
        fire(1, 1)

        def step(h, _):
            c = pl.multiple_of(h * 2, 2)
            drain(0)
            sum_store(c, 0, first=False)
            pl.when(c + 2 < s_out)(lambda: fire(c + 2, 0))
            drain(1)
            sum_store(c + 1, 1, first=False)
            pl.when(c + 3 < s_out)(lambda: fire(c + 3, 1))
            return 0

        drain(0)
        sum_store(0, 0, first=True)
        fire(2, 0)
        drain(1)
        sum_store(1, 1, first=True)
        fire(3, 1)
        lax.fori_loop(1, s_out // 2, step, 0)

        for b in range(2):
            for cp in out_copies(0, b):
                cp.wait()

    return k


def kernel(input_ids, tables):
    num_tables, vocab, d = tables.shape
    batch, seq = input_ids.shape
    s_out = seq // num_tables
    ids_flat = input_ids.reshape(-1)
    table_flat = tables.reshape(num_tables * vocab, d)
    raw = _build(batch, seq, num_tables, vocab, d)(ids_flat, table_flat)
    out = raw.reshape(s_out, d // TILE_R, batch // TILE_C, TILE_R, TILE_C)
    return out.transpose(2, 4, 0, 1, 3).reshape(batch, s_out, d)


# speedup vs baseline: 1.0318x; 1.0000x over previous
import functools

import jax
import jax.numpy as jnp
from jax import lax
from jax.experimental import pallas as pl
from jax.experimental.pallas import tpu as pltpu
from jax.experimental.pallas import tpu_sc as plsc

LANES = 16
SUB = 128
TILE_R = 8
TILE_C = 128


@functools.cache
def _build(batch, seq, num_tables, vocab, d):
    info = plsc.get_sparse_core_info()
    nc, ns = info.num_cores, info.num_subcores
    nw = nc * ns
    s_out = seq // num_tables
    rows_per_tile = batch // nw
    per_tile = rows_per_tile * seq
    chunk = rows_per_tile * num_tables
    n_sub = chunk // SUB
    dblks = d // TILE_R
    assert batch % (nw * TILE_C) == 0 and rows_per_tile == TILE_C
    assert seq % num_tables == 0 and d % LANES == 0 and chunk % SUB == 0
    assert LANES % num_tables == 0 and s_out % 2 == 0

    mesh = plsc.VectorSubcoreMesh(core_axis_name="c", subcore_axis_name="s")

    @functools.partial(
        pl.kernel,
        mesh=mesh,
        compiler_params=pltpu.CompilerParams(use_tc_tiling_on_sc=False, needs_layout_passes=False),
        out_type=jax.ShapeDtypeStruct(
            (s_out * d * batch,), jnp.float32),
        scratch_types=[
            pltpu.VMEM((per_tile,), jnp.int32),
            pltpu.VMEM((2, n_sub, SUB), jnp.int32),
            pltpu.VMEM((2, chunk, d), jnp.float32),
            pltpu.VMEM((2, dblks * TILE_R * TILE_C), jnp.float32),
            pltpu.SemaphoreType.DMA,
            pltpu.SemaphoreType.DMA,
            pltpu.SemaphoreType.DMA,
            pltpu.SemaphoreType.DMA,
        ],
    )
    def k(ids_hbm, table_hbm, out_hbm, ids_v, idx_v, rows_v, slab_v,
          gsem0, gsem1, osem0, osem1):
        wid = lax.axis_index("s") * nc + lax.axis_index("c")
        base = wid * per_tile
        gsems = (gsem0, gsem1)
        osems = (osem0, osem1)

        pltpu.sync_copy(ids_hbm.at[pl.ds(base, per_tile)], ids_v)
        iota = lax.iota(jnp.int32, LANES)
        pat_ids = (iota // num_tables) * seq + iota % num_tables
        offs = (iota % num_tables) * vocab
        rowp = iota * num_tables

        def fire(c, b):
            for v in range(chunk // LANES):
                p = v * LANES
                g = plsc.load_gather(
                    ids_v, [pat_ids + (v * LANES // num_tables) * seq
                            + c * num_tables])
                idx_v[b, p // SUB, pl.ds(p % SUB, LANES)] = g + offs
            for si in range(n_sub):
                pltpu.async_copy(
                    table_hbm.at[idx_v.at[b, si]],
                    rows_v.at[b, pl.ds(si * SUB, SUB)],
                    gsems[b],
                )

        def drain(b):
            for si in range(n_sub):
                pltpu.make_async_copy(
                    table_hbm.at[idx_v.at[b, si]],
                    rows_v.at[b, pl.ds(si * SUB, SUB)],
                    gsems[b],
                ).wait()

        tpc = TILE_R * TILE_C

        s_stride = d * batch
        db_stride = (batch // TILE_C) * tpc

        def out_copies(c, b):
            return [
                pltpu.make_async_copy(
                    slab_v.at[b, pl.ds(db * tpc, tpc)],
                    out_hbm.at[pl.ds(c * s_stride + db * db_stride
                                     + wid * tpc, tpc)],
                    osems[b],
                )
                for db in range(dblks)
            ]

        def sum_store(c, b, first):
            if not first:
                for cp in out_copies(c, b):
                    cp.wait()

            rows2 = rows_v.at[b]
            slab1 = slab_v.at[b]

            @plsc.parallel_loop(0, TILE_C // LANES, step=1, unroll=2)
            def body(g):
                b16 = pl.multiple_of(g * LANES, LANES)
                rowvs = [rowp + (b16 * num_tables + i)
                         for i in range(num_tables)]

                @plsc.parallel_loop(0, LANES, step=1, unroll=4)
                def rbody(r):
                    rot = (iota + r) % LANES
                    sp = ((rot // TILE_R) * (TILE_R * TILE_C)
                          + (rot % TILE_R) * TILE_C + iota)
                    for dd in range(d // LANES):
                        colv = rot + dd * LANES
                        posv = sp + (dd * 2 * tpc + b16)
                        acc = plsc.load_gather(rows2, [rowvs[0], colv])
                        for i in range(1, num_tables):
                            acc = acc + plsc.load_gather(
                                rows2, [rowvs[i], colv])
                        plsc.store_scatter(slab1, [posv], acc)
            for cp in out_copies(c, b):
                cp.start()

        fire(0, 0)
        fire(1, 1)

        def step(h, _):
            c = pl.multiple_of(h * 2, 2)
            drain(0)
            sum_store(c, 0, first=False)
            pl.when(c + 2 < s_out)(lambda: fire(c + 2, 0))
            drain(1)
            sum_store(c + 1, 1, first=False)
            pl.when(c + 3 < s_out)(lambda: fire(c + 3, 1))
            return 0

        drain(0)
        sum_store(0, 0, first=True)
        fire(2, 0)
        drain(1)
        sum_store(1, 1, first=True)
        fire(3, 1)
        lax.fori_loop(1, s_out // 2, step, 0)

        for b in range(2):
            for cp in out_copies(0, b):
                cp.wait()

    return k


def kernel(input_ids, tables):
    num_tables, vocab, d = tables.shape
    batch, seq = input_ids.shape
    s_out = seq // num_tables
    ids_flat = input_ids.reshape(-1)
    table_flat = tables.reshape(num_tables * vocab, d)
    raw = _build(batch, seq, num_tables, vocab, d)(ids_flat, table_flat)
    out = raw.reshape(s_out, d // TILE_R, batch // TILE_C, TILE_R, TILE_C)
    return out.transpose(2, 4, 0, 1, 3).reshape(batch, s_out, d)


# speedup vs baseline: 1.0319x; 1.0001x over previous
import functools

import jax
import jax.numpy as jnp
from jax import lax
from jax.experimental import pallas as pl
from jax.experimental.pallas import tpu as pltpu
from jax.experimental.pallas import tpu_sc as plsc

LANES = 16
SUB = 128
TILE_R = 8
TILE_C = 128


@functools.cache
def _build(batch, seq, num_tables, vocab, d):
    info = plsc.get_sparse_core_info()
    nc, ns = info.num_cores, info.num_subcores
    nw = nc * ns
    s_out = seq // num_tables
    rows_per_tile = batch // nw
    per_tile = rows_per_tile * seq
    chunk = rows_per_tile * num_tables
    n_sub = chunk // SUB
    dblks = d // TILE_R
    assert batch % (nw * TILE_C) == 0 and rows_per_tile == TILE_C
    assert seq % num_tables == 0 and d % LANES == 0 and chunk % SUB == 0
    assert LANES % num_tables == 0 and s_out % 2 == 0

    mesh = plsc.VectorSubcoreMesh(core_axis_name="c", subcore_axis_name="s")

    @functools.partial(
        pl.kernel,
        mesh=mesh,
        compiler_params=pltpu.CompilerParams(use_tc_tiling_on_sc=False, needs_layout_passes=False),
        out_type=jax.ShapeDtypeStruct(
            (s_out * d * batch,), jnp.float32),
        scratch_types=[
            pltpu.VMEM((per_tile,), jnp.int32),
            pltpu.VMEM((2, n_sub, SUB), jnp.int32),
            pltpu.VMEM((2, chunk, d), jnp.float32),
            pltpu.VMEM((2, dblks * TILE_R * TILE_C), jnp.float32),
            pltpu.SemaphoreType.DMA,
            pltpu.SemaphoreType.DMA,
            pltpu.SemaphoreType.DMA,
            pltpu.SemaphoreType.DMA,
        ],
    )
    def k(ids_hbm, table_hbm, out_hbm, ids_v, idx_v, rows_v, slab_v,
          gsem0, gsem1, osem0, osem1):
        wid = lax.axis_index("s") * nc + lax.axis_index("c")
        base = wid * per_tile
        gsems = (gsem0, gsem1)
        osems = (osem0, osem1)

        pltpu.sync_copy(ids_hbm.at[pl.ds(base, per_tile)], ids_v)
        iota = lax.iota(jnp.int32, LANES)
        pat_ids = (iota // num_tables) * seq + iota % num_tables
        offs = (iota % num_tables) * vocab
        rowp = iota * num_tables

        def fire(c, b):
            for v in range(chunk // LANES):
                p = v * LANES
                g = plsc.load_gather(
                    ids_v, [pat_ids + (v * LANES // num_tables) * seq
                            + c * num_tables])
                idx_v[b, p // SUB, pl.ds(p % SUB, LANES)] = g + offs
            for si in range(n_sub):
                pltpu.async_copy(
                    table_hbm.at[idx_v.at[b, si]],
                    rows_v.at[b, pl.ds(si * SUB, SUB)],
                    gsems[b],
                )

        def drain(b):
            for si in range(n_sub):
                pltpu.make_async_copy(
                    table_hbm.at[idx_v.at[b, si]],
                    rows_v.at[b, pl.ds(si * SUB, SUB)],
                    gsems[b],
                ).wait()

        tpc = TILE_R * TILE_C

        s_stride = d * batch
        db_stride = (batch // TILE_C) * tpc

        def out_copies(c, b):
            return [
                pltpu.make_async_copy(
                    slab_v.at[b, pl.ds(db * tpc, tpc)],
                    out_hbm.at[pl.ds(c * s_stride + db * db_stride
                                     + wid * tpc, tpc)],
                    osems[b],
                )
                for db in range(dblks)
            ]

        def sum_store(c, b, first):
            if not first:
                for cp in out_copies(c, b):
                    cp.wait()

            rows2 = rows_v.at[b]
            slab1 = slab_v.at[b]

            @plsc.parallel_loop(0, TILE_C // LANES, step=1)
            def body(g):
                b16 = pl.multiple_of(g * LANES, LANES)
                rowvs = [rowp + (b16 * num_tables + i)
                         for i in range(num_tables)]

                @plsc.parallel_loop(0, LANES, step=1, unroll=4)
                def rbody(r):
                    rot = (iota + r) % LANES
                    sp = ((rot // TILE_R) * (TILE_R * TILE_C)
                          + (rot % TILE_R) * TILE_C + iota)
                    for dd in range(d // LANES):
                        colv = rot + dd * LANES
                        posv = sp + (dd * 2 * tpc + b16)
                        acc = plsc.load_gather(rows2, [rowvs[0], colv])
                        for i in range(1, num_tables):
                            acc = acc + plsc.load_gather(
                                rows2, [rowvs[i], colv])
                        plsc.store_scatter(slab1, [posv], acc)
            for cp in out_copies(c, b):
                cp.start()

        fire(0, 0)
        fire(1, 1)

        def step(h, _):
            c = pl.multiple_of(h * 2, 2)
            drain(0)
            sum_store(c, 0, first=False)
            pl.when(c + 2 < s_out)(lambda: fire(c + 2, 0))
            drain(1)
            sum_store(c + 1, 1, first=False)
            pl.when(c + 3 < s_out)(lambda: fire(c + 3, 1))
            return 0

        drain(0)
        sum_store(0, 0, first=True)
        fire(2, 0)
        drain(1)
        sum_store(1, 1, first=True)
        fire(3, 1)
        lax.fori_loop(1, s_out // 2, step, 0)

        for b in range(2):
            for cp in out_copies(0, b):
                cp.wait()

    return k


def kernel(input_ids, tables):
    num_tables, vocab, d = tables.shape
    batch, seq = input_ids.shape
    s_out = seq // num_tables
    ids_flat = input_ids.reshape(-1)
    table_flat = tables.reshape(num_tables * vocab, d)
    raw = _build(batch, seq, num_tables, vocab, d)(ids_flat, table_flat)
    out = raw.reshape(s_out, d // TILE_R, batch // TILE_C, TILE_R, TILE_C)
    return out.transpose(2, 4, 0, 1, 3).reshape(batch, s_out, d)
